# Initial kernel scaffold; baseline (speedup 1.0000x reference)
import functools

import jax
import jax.numpy as jnp
from jax import lax
from jax.experimental import pallas as pl
from jax.experimental.pallas import tpu as pltpu
from jax.experimental.pallas import tpu_sc as plsc

NB = 4096
HL = 200
ED = 128
NC = 2
NS = 16
NW = NC * NS
PER = NB // NW
SPLIT_A = 104
SPLIT_B = HL - SPLIT_A

_mesh = plsc.VectorSubcoreMesh(core_axis_name="c", subcore_axis_name="s")


@functools.partial(
    pl.kernel,
    out_type=jax.ShapeDtypeStruct((NB, ED, HL), jnp.float32),
    mesh=_mesh,
    scratch_types=[
        pltpu.VMEM((SPLIT_A,), jnp.int32),
        pltpu.VMEM((SPLIT_B,), jnp.int32),
        pltpu.VMEM((HL, ED), jnp.float32),
        pltpu.VMEM((ED, HL), jnp.float32),
        pltpu.SemaphoreType.DMA,
    ],
)
def _embed_permute(x_hbm, table_hbm, out_hbm, idx_a, idx_b, gbuf, tbuf, sem):
    wid = lax.axis_index("s") * NC + lax.axis_index("c")
    iota = lax.iota(jnp.int32, 16)
    d_idx = [iota + (16 * k) for k in range(ED // 16)]

    def per_batch(i, carry):
        b = wid * PER + i
        pltpu.sync_copy(x_hbm.at[b, pl.ds(0, SPLIT_A)], idx_a)
        pltpu.sync_copy(x_hbm.at[b, pl.ds(SPLIT_A, SPLIT_B)], idx_b)
        ca = pltpu.async_copy(table_hbm.at[idx_a], gbuf.at[pl.ds(0, SPLIT_A)], sem)
        cb = pltpu.async_copy(table_hbm.at[idx_b], gbuf.at[pl.ds(SPLIT_A, SPLIT_B)], sem)
        ca.wait()
        cb.wait()

        def per_l(l, c2):
            l_vec = jnp.zeros((16,), jnp.int32) + l
            for k in range(ED // 16):
                v = gbuf[l, pl.ds(16 * k, 16)]
                plsc.store_scatter(tbuf, [d_idx[k], l_vec], v)
            return c2

        lax.fori_loop(0, HL, per_l, 0)
        pltpu.sync_copy(tbuf, out_hbm.at[b])
        return carry

    lax.fori_loop(0, PER, per_batch, 0)


def kernel(x, table):
    return _embed_permute(x.astype(jnp.int32), table)


# speedup vs baseline: 1.2650x; 1.2650x over previous
import functools

import jax
import jax.numpy as jnp
from jax import lax
from jax.experimental import pallas as pl
from jax.experimental.pallas import tpu as pltpu
from jax.experimental.pallas import tpu_sc as plsc

NB = 4096
HL = 200
ED = 128
NC = 2
NS = 16
NW = NC * NS
PER = NB // NW
SPLIT_A = 104
SPLIT_B = HL - SPLIT_A

_mesh = plsc.VectorSubcoreMesh(core_axis_name="c", subcore_axis_name="s")


@functools.partial(
    pl.kernel,
    out_type=jax.ShapeDtypeStruct((NB * ED * HL,), jnp.float32),
    mesh=_mesh,
    scratch_types=[
        pltpu.VMEM((SPLIT_A,), jnp.int32),
        pltpu.VMEM((SPLIT_B,), jnp.int32),
        pltpu.VMEM((HL, ED), jnp.float32),
        pltpu.VMEM((ED * HL,), jnp.float32),
        pltpu.SemaphoreType.DMA,
    ],
    compiler_params=pltpu.CompilerParams(
        use_tc_tiling_on_sc=False, needs_layout_passes=False
    ),
)
def _embed_permute(x_hbm, table_hbm, out_hbm, idx_a, idx_b, gbuf, tbuf, sem):
    wid = lax.axis_index("s") * NC + lax.axis_index("c")
    iota = lax.iota(jnp.int32, 16)
    d_base = [iota * HL + (16 * k * HL) for k in range(ED // 16)]

    def per_batch(i, carry):
        b = wid * PER + i
        base = pl.multiple_of(b * HL, 8)
        pltpu.sync_copy(x_hbm.at[pl.ds(base, SPLIT_A)], idx_a)
        pltpu.sync_copy(x_hbm.at[pl.ds(base + SPLIT_A, SPLIT_B)], idx_b)
        ca = pltpu.async_copy(table_hbm.at[idx_a], gbuf.at[pl.ds(0, SPLIT_A)], sem)
        cb = pltpu.async_copy(table_hbm.at[idx_b], gbuf.at[pl.ds(SPLIT_A, SPLIT_B)], sem)
        ca.wait()
        cb.wait()

        def per_l(l, c2):
            l_vec = jnp.zeros((16,), jnp.int32) + l
            for k in range(ED // 16):
                v = gbuf[l, pl.ds(16 * k, 16)]
                plsc.store_scatter(tbuf, [d_base[k] + l_vec], v)
            return c2

        lax.fori_loop(0, HL, per_l, 0)
        obase = pl.multiple_of(b * (ED * HL), 8)
        pltpu.sync_copy(tbuf, out_hbm.at[pl.ds(obase, ED * HL)])
        return carry

    lax.fori_loop(0, PER, per_batch, 0)


def kernel(x, table):
    flat = _embed_permute(x.astype(jnp.int32).reshape(-1), table)
    return flat.reshape(NB, ED, HL)


# speedup vs baseline: 1.5810x; 1.2497x over previous
import functools

import jax
import jax.numpy as jnp
from jax import lax
from jax.experimental import pallas as pl
from jax.experimental.pallas import tpu as pltpu
from jax.experimental.pallas import tpu_sc as plsc

NB = 4096
HL = 200
ED = 128
NC = 2
NS = 16
NW = NC * NS
PER = NB // NW
IDX_WORDS = PER * HL
OUT_WORDS = ED * HL
SPLIT_A = 104
SPLIT_B = HL - SPLIT_A

_mesh = plsc.VectorSubcoreMesh(core_axis_name="c", subcore_axis_name="s")


@functools.partial(
    pl.kernel,
    out_type=jax.ShapeDtypeStruct((NB * ED * HL,), jnp.float32),
    mesh=_mesh,
    scratch_types=[
        pltpu.VMEM((IDX_WORDS,), jnp.int32),
        pltpu.VMEM((HL, ED), jnp.float32),
        pltpu.VMEM((HL, ED), jnp.float32),
        pltpu.VMEM((OUT_WORDS,), jnp.float32),
        pltpu.VMEM((OUT_WORDS,), jnp.float32),
        pltpu.SemaphoreType.DMA,
        pltpu.SemaphoreType.DMA,
        pltpu.SemaphoreType.DMA,
        pltpu.SemaphoreType.DMA,
    ],
    compiler_params=pltpu.CompilerParams(
        use_tc_tiling_on_sc=False, needs_layout_passes=False
    ),
)
def _embed_permute(x_hbm, table_hbm, out_hbm, idx_buf, g0, g1, t0, t1,
                   sg0, sg1, sw0, sw1):
    wid = lax.axis_index("s") * NC + lax.axis_index("c")
    iota = lax.iota(jnp.int32, 16)
    d_base = [iota * HL + (16 * k * HL) for k in range(ED // 16)]

    xbase = pl.multiple_of(wid * IDX_WORDS, 8)
    pltpu.sync_copy(x_hbm.at[pl.ds(xbase, IDX_WORDS)], idx_buf)

    def gather_copies(b, gbuf, sem):
        o = pl.multiple_of(b * HL, 8)
        ca = pltpu.make_async_copy(
            table_hbm.at[idx_buf.at[pl.ds(o, SPLIT_A)]],
            gbuf.at[pl.ds(0, SPLIT_A)], sem)
        cb = pltpu.make_async_copy(
            table_hbm.at[idx_buf.at[pl.ds(o + SPLIT_A, SPLIT_B)]],
            gbuf.at[pl.ds(SPLIT_A, SPLIT_B)], sem)
        return ca, cb

    def out_copy(b, tbuf, sem):
        ob = pl.multiple_of((wid * PER + b) * OUT_WORDS, 8)
        return pltpu.make_async_copy(tbuf, out_hbm.at[pl.ds(ob, OUT_WORDS)], sem)

    def transpose(gbuf, tbuf):
        def per_l(l, c2):
            lv = jnp.zeros((16,), jnp.int32) + l
            for k in range(ED // 16):
                v = gbuf[l, pl.ds(16 * k, 16)]
                plsc.store_scatter(tbuf, [d_base[k] + lv], v)
            return c2
        lax.fori_loop(0, HL, per_l, 0, unroll=2)

    for c in gather_copies(0, g0, sg0):
        c.start()

    def phase(b, gcur, scur, gnxt, snxt, tcur, swcur):
        for c in gather_copies(b, gcur, scur):
            c.wait()

        @pl.when(b + 1 < PER)
        def _():
            for c in gather_copies(b + 1, gnxt, snxt):
                c.start()

        @pl.when(b >= 2)
        def _():
            out_copy(b - 2, tcur, swcur).wait()

        transpose(gcur, tcur)
        out_copy(b, tcur, swcur).start()

    def iter_body(i, carry):
        b0 = 2 * i
        phase(b0, g0, sg0, g1, sg1, t0, sw0)
        phase(b0 + 1, g1, sg1, g0, sg0, t1, sw1)
        return carry

    lax.fori_loop(0, PER // 2, iter_body, 0)

    out_copy(PER - 2, t0, sw0).wait()
    out_copy(PER - 1, t1, sw1).wait()


def kernel(x, table):
    flat = _embed_permute(x.astype(jnp.int32).reshape(-1), table)
    return flat.reshape(NB, ED, HL)
